# baseline (speedup 1.0000x reference)
import functools

import jax
import jax.numpy as jnp
from jax import lax
from jax.experimental import pallas as pl
from jax.experimental.pallas import tpu as pltpu
from jax.experimental.pallas import tpu_sc as plsc

VOCAB = 100000
EMB_DIM = 128
B, L = 4096, 200
HALF = VOCAB // 2

GRID_T = 5
ROWS_T = HALF // GRID_T

NC, NS = 2, 16
NW = NC * NS
ROWS_W = B // NW
GROUPS_W = ROWS_W // 16
UNROLL = 25
TW_PAD = HALF + 16


def _tc_body(w1_ref, w2_ref, b1_ref, b2_ref, tla_ref, tlb_ref, t_ref, c_ref):
    v = jnp.dot(w2_ref[...], w1_ref[...], preferred_element_type=jnp.float32)
    dims = (((1,), (1,)), ((), ()))
    ta = lax.dot_general(v, tla_ref[...], dims,
                         preferred_element_type=jnp.float32)
    tb = lax.dot_general(v, tlb_ref[...], dims,
                         preferred_element_type=jnp.float32)
    ba = lax.bitcast_convert_type(ta, jnp.int32)
    bb = lax.bitcast_convert_type(tb, jnp.int32)
    t_ref[0, :, :] = ((ba >> 16) & 0xFFFF) | (bb & jnp.int32(-65536))
    c = jnp.sum(w2_ref[...] * b1_ref[...]) + b2_ref[0, 0]
    c_ref[...] = jnp.full((1, 128), c, jnp.float32)


def _tc_stage(emb_table, W1, b1, W2, b2):
    tw3, c_out = pl.pallas_call(
        _tc_body,
        grid=(GRID_T,),
        in_specs=[
            pl.BlockSpec((64, EMB_DIM), lambda i: (0, 0)),
            pl.BlockSpec((1, 64), lambda i: (0, 0)),
            pl.BlockSpec((1, 64), lambda i: (0, 0)),
            pl.BlockSpec((1, 1), lambda i: (0, 0)),
            pl.BlockSpec((ROWS_T, EMB_DIM), lambda i: (i, 0)),
            pl.BlockSpec((ROWS_T, EMB_DIM), lambda i: (i + GRID_T, 0)),
        ],
        out_specs=[
            pl.BlockSpec((1, 1, ROWS_T), lambda i: (i, 0, 0)),
            pl.BlockSpec((1, 128), lambda i: (0, 0)),
        ],
        out_shape=[
            jax.ShapeDtypeStruct((GRID_T, 1, ROWS_T), jnp.int32),
            jax.ShapeDtypeStruct((1, 128), jnp.float32),
        ],
    )(W1, W2, b1.reshape(1, 64), b2.reshape(1, 1), emb_table, emb_table)
    return tw3.reshape(HALF), c_out.reshape(128)


_SC_MESH = plsc.VectorSubcoreMesh(
    core_axis_name="c", subcore_axis_name="s", num_cores=NC, num_subcores=NS
)


@functools.partial(
    pl.kernel,
    out_type=jax.ShapeDtypeStruct((B,), jnp.float32),
    mesh=_SC_MESH,
    compiler_params=pltpu.CompilerParams(needs_layout_passes=False),
    scratch_types=[
        pltpu.VMEM((TW_PAD,), jnp.int32),
        pltpu.VMEM((L * ROWS_W,), jnp.int32),
        pltpu.VMEM((ROWS_W,), jnp.float32),
        pltpu.VMEM((16,), jnp.float32),
        pltpu.VMEM_SHARED((HALF,), jnp.int32),
        pltpu.SemaphoreType.DMA,
        pltpu.SemaphoreType.DMA,
        pltpu.SemaphoreType.DMA,
    ],
)
def _sc_pool(tw_hbm, xm_hbm, c_hbm, out_hbm, tw_v, xv, o_v, c_v, ts_sh,
             sem_t, sem_x, sem_c):
    wid = lax.axis_index("s") * NC + lax.axis_index("c")
    row0 = wid * ROWS_W
    base = wid * (L * ROWS_W)
    zero16i = jnp.zeros((16,), jnp.int32)
    zero16 = jnp.zeros((16,), jnp.float32)
    one16 = jnp.ones((16,), jnp.float32)
    half16 = jnp.full((16,), HALF, jnp.int32)
    himask = jnp.full((16,), -65536, jnp.int32)
    lomask = jnp.full((16,), 0x7FFFFFFF, jnp.int32)

    with jax.named_scope("sc_copy"):
        cx = pltpu.async_copy(
            xm_hbm.at[pl.ds(base, L * ROWS_W)], xv, sem_x)
        cc = pltpu.async_copy(c_hbm.at[pl.ds(0, 16)], c_v, sem_c)

        @pl.when(lax.axis_index("s") == 0)
        def _():
            pltpu.sync_copy(tw_hbm, ts_sh)

        plsc.subcore_barrier()
        ct = pltpu.async_copy(ts_sh, tw_v.at[pl.ds(0, HALF)], sem_t)
        tw_v[pl.ds(HALF, 16)] = zero16i
        ct.wait()
        cx.wait()
        cc.wait()

    c16 = c_v[...]

    with jax.named_scope("sc_compute"):
        def group(g, _):
            goff = g * 16

            def body(kk, carry):
                acc, mac = carry
                for u in range(UNROLL):
                    xi = xv[pl.ds((kk * UNROLL + u) * ROWS_W + goff, 16)]
                    hi = xi < zero16i
                    wi = xi & lomask
                    w = plsc.load_gather(tw_v, [wi])
                    bits = jnp.where(hi, w & himask, w << 16)
                    acc = acc + plsc.bitcast(bits, jnp.float32)
                    mac = mac + jnp.where(wi < half16, one16, zero16)
                return acc, mac

            acc, mac = lax.fori_loop(0, L // UNROLL, body, (zero16, zero16))
            z = acc / jnp.maximum(mac, one16) + c16
            o_v[pl.ds(goff, 16)] = one16 / (one16 + jnp.exp(-z))
            return 0

        lax.fori_loop(0, GROUPS_W, group, 0)

    pltpu.sync_copy(o_v, out_hbm.at[pl.ds(row0, ROWS_W)])


def kernel(x, attention_mask, emb_table, W1, b1, W2, b2):
    tw, c_vec = _tc_stage(emb_table, W1, b1, W2, b2)
    xm = jnp.where(attention_mask != 0, x.astype(jnp.int32), VOCAB)
    xm = jnp.where(xm >= HALF, (xm - HALF) | jnp.int32(-(2**31)), xm)
    xm = xm.reshape(NW, ROWS_W, L).swapaxes(1, 2)
    return _sc_pool(tw, xm.reshape(B * L), c_vec)


# speedup vs baseline: 1.1063x; 1.1063x over previous
import functools

import jax
import jax.numpy as jnp
from jax import lax
from jax.experimental import pallas as pl
from jax.experimental.pallas import tpu as pltpu
from jax.experimental.pallas import tpu_sc as plsc

VOCAB = 100000
EMB_DIM = 128
B, L = 4096, 200
HALF = VOCAB // 2

GRID_T = 5
ROWS_T = HALF // GRID_T

NC, NS = 2, 16
NW = NC * NS
ROWS_W = B // NW
GROUPS_W = ROWS_W // 16
UNROLL = 4
TW_PAD = HALF + 16


def _tc_body(w1_ref, w2_ref, b1_ref, b2_ref, tla_ref, tlb_ref, t_ref, c_ref):
    v = jnp.dot(w2_ref[...], w1_ref[...], preferred_element_type=jnp.float32)
    dims = (((1,), (1,)), ((), ()))
    ta = lax.dot_general(v, tla_ref[...], dims,
                         preferred_element_type=jnp.float32)
    tb = lax.dot_general(v, tlb_ref[...], dims,
                         preferred_element_type=jnp.float32)
    ba = lax.bitcast_convert_type(ta, jnp.int32)
    bb = lax.bitcast_convert_type(tb, jnp.int32)
    t_ref[0, :, :] = ((ba >> 16) & 0xFFFF) | (bb & jnp.int32(-65536))
    c = jnp.sum(w2_ref[...] * b1_ref[...]) + b2_ref[0, 0]
    c_ref[...] = jnp.full((1, 128), c, jnp.float32)


def _tc_stage(emb_table, W1, b1, W2, b2):
    tw3, c_out = pl.pallas_call(
        _tc_body,
        grid=(GRID_T,),
        in_specs=[
            pl.BlockSpec((64, EMB_DIM), lambda i: (0, 0)),
            pl.BlockSpec((1, 64), lambda i: (0, 0)),
            pl.BlockSpec((1, 64), lambda i: (0, 0)),
            pl.BlockSpec((1, 1), lambda i: (0, 0)),
            pl.BlockSpec((ROWS_T, EMB_DIM), lambda i: (i, 0)),
            pl.BlockSpec((ROWS_T, EMB_DIM), lambda i: (i + GRID_T, 0)),
        ],
        out_specs=[
            pl.BlockSpec((1, 1, ROWS_T), lambda i: (i, 0, 0)),
            pl.BlockSpec((1, 128), lambda i: (0, 0)),
        ],
        out_shape=[
            jax.ShapeDtypeStruct((GRID_T, 1, ROWS_T), jnp.int32),
            jax.ShapeDtypeStruct((1, 128), jnp.float32),
        ],
    )(W1, W2, b1.reshape(1, 64), b2.reshape(1, 1), emb_table, emb_table)
    return tw3.reshape(HALF), c_out.reshape(128)


_SC_MESH = plsc.VectorSubcoreMesh(
    core_axis_name="c", subcore_axis_name="s", num_cores=NC, num_subcores=NS
)


@functools.partial(
    pl.kernel,
    out_type=jax.ShapeDtypeStruct((B,), jnp.float32),
    mesh=_SC_MESH,
    compiler_params=pltpu.CompilerParams(needs_layout_passes=False),
    scratch_types=[
        pltpu.VMEM((TW_PAD,), jnp.int32),
        pltpu.VMEM((L * ROWS_W,), jnp.int32),
        pltpu.VMEM((ROWS_W,), jnp.float32),
        pltpu.VMEM((16,), jnp.float32),
        pltpu.VMEM_SHARED((HALF,), jnp.int32),
        pltpu.SemaphoreType.DMA,
        pltpu.SemaphoreType.DMA,
        pltpu.SemaphoreType.DMA,
    ],
)
def _sc_pool(tw_hbm, xm_hbm, c_hbm, out_hbm, tw_v, xv, o_v, c_v, ts_sh,
             sem_t, sem_x, sem_c):
    wid = lax.axis_index("s") * NC + lax.axis_index("c")
    row0 = wid * ROWS_W
    base = wid * (L * ROWS_W)
    zero16i = jnp.zeros((16,), jnp.int32)
    zero16 = jnp.zeros((16,), jnp.float32)
    one16 = jnp.ones((16,), jnp.float32)
    half16 = jnp.full((16,), HALF, jnp.int32)
    himask = jnp.full((16,), -65536, jnp.int32)
    lomask = jnp.full((16,), 0x7FFFFFFF, jnp.int32)

    with jax.named_scope("sc_copy"):
        cx = pltpu.async_copy(
            xm_hbm.at[pl.ds(base, L * ROWS_W)], xv, sem_x)
        cc = pltpu.async_copy(c_hbm.at[pl.ds(0, 16)], c_v, sem_c)

        @pl.when(lax.axis_index("s") == 0)
        def _():
            pltpu.sync_copy(tw_hbm, ts_sh)

        plsc.subcore_barrier()
        ct = pltpu.async_copy(ts_sh, tw_v.at[pl.ds(0, HALF)], sem_t)
        tw_v[pl.ds(HALF, 16)] = zero16i
        ct.wait()
        cx.wait()
        cc.wait()

    c16 = c_v[...]

    with jax.named_scope("sc_compute"):
        def group(g, _):
            goff = g * 16

            def body(kk, carry):
                acc, mac = carry
                for u in range(UNROLL):
                    xi = xv[pl.ds((kk * UNROLL + u) * ROWS_W + goff, 16)]
                    hi = xi < zero16i
                    wi = xi & lomask
                    w = plsc.load_gather(tw_v, [wi])
                    bits = jnp.where(hi, w & himask, w << 16)
                    acc = acc + plsc.bitcast(bits, jnp.float32)
                    mac = mac + jnp.where(wi < half16, one16, zero16)
                return acc, mac

            acc, mac = lax.fori_loop(0, L // UNROLL, body, (zero16, zero16))
            z = acc / jnp.maximum(mac, one16) + c16
            o_v[pl.ds(goff, 16)] = one16 / (one16 + jnp.exp(-z))
            return 0

        lax.fori_loop(0, GROUPS_W, group, 0)

    pltpu.sync_copy(o_v, out_hbm.at[pl.ds(row0, ROWS_W)])


def kernel(x, attention_mask, emb_table, W1, b1, W2, b2):
    tw, c_vec = _tc_stage(emb_table, W1, b1, W2, b2)
    xm = jnp.where(attention_mask != 0, x.astype(jnp.int32), VOCAB)
    xm = jnp.where(xm >= HALF, (xm - HALF) | jnp.int32(-(2**31)), xm)
    xm = xm.reshape(NW, ROWS_W, L).swapaxes(1, 2)
    return _sc_pool(tw, xm.reshape(B * L), c_vec)


# speedup vs baseline: 1.1096x; 1.0030x over previous
import functools

import jax
import jax.numpy as jnp
from jax import lax
from jax.experimental import pallas as pl
from jax.experimental.pallas import tpu as pltpu
from jax.experimental.pallas import tpu_sc as plsc

VOCAB = 100000
EMB_DIM = 128
B, L = 4096, 200
HALF = VOCAB // 2

GRID_T = 5
ROWS_T = HALF // GRID_T

NC, NS = 2, 16
NW = NC * NS
ROWS_W = B // NW
GROUPS_W = ROWS_W // 16
UNROLL = 8
TW_PAD = HALF + 16


def _tc_body(w1_ref, w2_ref, b1_ref, b2_ref, tla_ref, tlb_ref, t_ref, c_ref):
    v = jnp.dot(w2_ref[...], w1_ref[...], preferred_element_type=jnp.float32)
    dims = (((1,), (1,)), ((), ()))
    ta = lax.dot_general(v, tla_ref[...], dims,
                         preferred_element_type=jnp.float32)
    tb = lax.dot_general(v, tlb_ref[...], dims,
                         preferred_element_type=jnp.float32)
    ba = lax.bitcast_convert_type(ta, jnp.int32)
    bb = lax.bitcast_convert_type(tb, jnp.int32)
    t_ref[0, :, :] = ((ba >> 16) & 0xFFFF) | (bb & jnp.int32(-65536))
    c = jnp.sum(w2_ref[...] * b1_ref[...]) + b2_ref[0, 0]
    c_ref[...] = jnp.full((1, 128), c, jnp.float32)


def _tc_stage(emb_table, W1, b1, W2, b2):
    tw3, c_out = pl.pallas_call(
        _tc_body,
        grid=(GRID_T,),
        in_specs=[
            pl.BlockSpec((64, EMB_DIM), lambda i: (0, 0)),
            pl.BlockSpec((1, 64), lambda i: (0, 0)),
            pl.BlockSpec((1, 64), lambda i: (0, 0)),
            pl.BlockSpec((1, 1), lambda i: (0, 0)),
            pl.BlockSpec((ROWS_T, EMB_DIM), lambda i: (i, 0)),
            pl.BlockSpec((ROWS_T, EMB_DIM), lambda i: (i + GRID_T, 0)),
        ],
        out_specs=[
            pl.BlockSpec((1, 1, ROWS_T), lambda i: (i, 0, 0)),
            pl.BlockSpec((1, 128), lambda i: (0, 0)),
        ],
        out_shape=[
            jax.ShapeDtypeStruct((GRID_T, 1, ROWS_T), jnp.int32),
            jax.ShapeDtypeStruct((1, 128), jnp.float32),
        ],
    )(W1, W2, b1.reshape(1, 64), b2.reshape(1, 1), emb_table, emb_table)
    return tw3.reshape(HALF), c_out.reshape(128)


_SC_MESH = plsc.VectorSubcoreMesh(
    core_axis_name="c", subcore_axis_name="s", num_cores=NC, num_subcores=NS
)


@functools.partial(
    pl.kernel,
    out_type=jax.ShapeDtypeStruct((B,), jnp.float32),
    mesh=_SC_MESH,
    compiler_params=pltpu.CompilerParams(needs_layout_passes=False),
    scratch_types=[
        pltpu.VMEM((TW_PAD,), jnp.int32),
        pltpu.VMEM((L * ROWS_W,), jnp.int32),
        pltpu.VMEM((ROWS_W,), jnp.float32),
        pltpu.VMEM((16,), jnp.float32),
        pltpu.VMEM_SHARED((HALF,), jnp.int32),
        pltpu.SemaphoreType.DMA,
        pltpu.SemaphoreType.DMA,
        pltpu.SemaphoreType.DMA,
    ],
)
def _sc_pool(tw_hbm, xm_hbm, c_hbm, out_hbm, tw_v, xv, o_v, c_v, ts_sh,
             sem_t, sem_x, sem_c):
    wid = lax.axis_index("s") * NC + lax.axis_index("c")
    row0 = wid * ROWS_W
    base = wid * (L * ROWS_W)
    zero16i = jnp.zeros((16,), jnp.int32)
    zero16 = jnp.zeros((16,), jnp.float32)
    one16 = jnp.ones((16,), jnp.float32)
    half16 = jnp.full((16,), HALF, jnp.int32)
    himask = jnp.full((16,), -65536, jnp.int32)
    lomask = jnp.full((16,), 0x7FFFFFFF, jnp.int32)

    with jax.named_scope("sc_copy"):
        cx = pltpu.async_copy(
            xm_hbm.at[pl.ds(base, L * ROWS_W)], xv, sem_x)
        cc = pltpu.async_copy(c_hbm.at[pl.ds(0, 16)], c_v, sem_c)

        @pl.when(lax.axis_index("s") == 0)
        def _():
            pltpu.sync_copy(tw_hbm, ts_sh)

        plsc.subcore_barrier()
        ct = pltpu.async_copy(ts_sh, tw_v.at[pl.ds(0, HALF)], sem_t)
        tw_v[pl.ds(HALF, 16)] = zero16i
        ct.wait()
        cx.wait()
        cc.wait()

    c16 = c_v[...]

    with jax.named_scope("sc_compute"):
        def group(g, _):
            goff = g * 16

            def body(kk, carry):
                acc, mac = carry
                for u in range(UNROLL):
                    xi = xv[pl.ds((kk * UNROLL + u) * ROWS_W + goff, 16)]
                    hi = xi < zero16i
                    wi = xi & lomask
                    w = plsc.load_gather(tw_v, [wi])
                    bits = jnp.where(hi, w & himask, w << 16)
                    acc = acc + plsc.bitcast(bits, jnp.float32)
                    mac = mac + jnp.where(wi < half16, one16, zero16)
                return acc, mac

            acc, mac = lax.fori_loop(0, L // UNROLL, body, (zero16, zero16))
            z = acc / jnp.maximum(mac, one16) + c16
            o_v[pl.ds(goff, 16)] = one16 / (one16 + jnp.exp(-z))
            return 0

        lax.fori_loop(0, GROUPS_W, group, 0)

    pltpu.sync_copy(o_v, out_hbm.at[pl.ds(row0, ROWS_W)])


def kernel(x, attention_mask, emb_table, W1, b1, W2, b2):
    tw, c_vec = _tc_stage(emb_table, W1, b1, W2, b2)
    xm = jnp.where(attention_mask != 0, x.astype(jnp.int32), VOCAB)
    xm = jnp.where(xm >= HALF, (xm - HALF) | jnp.int32(-(2**31)), xm)
    xm = xm.reshape(NW, ROWS_W, L).swapaxes(1, 2)
    return _sc_pool(tw, xm.reshape(B * L), c_vec)
